# baseline (speedup 1.0000x reference)
import jax
import jax.numpy as jnp
from jax import lax
from jax.experimental import pallas as pl
from jax.experimental.pallas import tpu as pltpu
from jax.experimental.pallas import tpu_sc as plsc

N_NODES = 10000
N_EDGES = 320000
D = 128

NC = 2
NS = 16
NW = NC * NS

CHUNK = 128
SB = 8
NSB = 10
NCHUNK = SB * NSB
E_PAD = NW * NCHUNK * CHUNK
NPAD = 10112
ROWS_PER_TILE = NPAD // NS


def _scatter_gather_kernel(x_hbm, src_hbm, dst_hbm, zero_hbm, h2_hbm,
                           sidx0, sidx1, didx0, didx1, buf0, buf1, acc,
                           gsem0, gsem1):
    c = lax.axis_index("c")
    s = lax.axis_index("s")
    wid = s * NC + c

    sidx = (sidx0, sidx1)
    didx = (didx0, didx1)
    bufs = (buf0, buf1)
    gsem = (gsem0, gsem1)

    def gather(idx_row, b):
        return pltpu.async_copy(x_hbm.at[idx_row], bufs[b], gsem[b])

    def gather_wait(b):
        pltpu.make_async_copy(x_hbm.at[sidx0.at[0]], bufs[b], gsem[b]).wait()

    r0 = s * ROWS_PER_TILE
    pltpu.sync_copy(zero_hbm.at[pl.ds(r0, ROWS_PER_TILE)],
                    acc.at[pl.ds(r0, ROWS_PER_TILE)])

    pltpu.sync_copy(src_hbm.at[wid, 0], sidx0)
    pltpu.sync_copy(dst_hbm.at[wid, 0], didx0)
    plsc.subcore_barrier()
    gather(sidx0.at[0], 0)
    gather(sidx0.at[1], 1)

    def body(i, carry):
        for sbk in range(2):
            sb = 2 * i + sbk
            sbn = lax.rem(sb + 1, NSB)
            pltpu.sync_copy(src_hbm.at[wid, sbn], sidx[1 - sbk])
            pltpu.sync_copy(dst_hbm.at[wid, sbn], didx[1 - sbk])
            for k in range(SB):
                b = k % 2
                gather_wait(b)
                pltpu.sync_copy(bufs[b], acc.at[didx[sbk].at[k]], add=True)
                if k < SB - 2:
                    gather(sidx[sbk].at[k + 2], b)
                else:
                    gather(sidx[1 - sbk].at[k + 2 - SB], b)
        return carry

    lax.fori_loop(0, NSB // 2, body, 0)

    gather_wait(0)
    gather_wait(1)

    plsc.subcore_barrier()
    pltpu.sync_copy(acc.at[pl.ds(r0, ROWS_PER_TILE)],
                    h2_hbm.at[c, pl.ds(r0, ROWS_PER_TILE)])


@jax.jit
def _segment_sum_sc(x, src, dst, zero):
    mesh = plsc.VectorSubcoreMesh(core_axis_name="c", subcore_axis_name="s")
    return pl.kernel(
        _scatter_gather_kernel,
        out_type=jax.ShapeDtypeStruct((NC, NPAD, D), jnp.float32),
        mesh=mesh,
        compiler_params=pltpu.CompilerParams(use_tc_tiling_on_sc=False),
        scratch_types=[
            pltpu.VMEM((SB, CHUNK), jnp.int32),
            pltpu.VMEM((SB, CHUNK), jnp.int32),
            pltpu.VMEM((SB, CHUNK), jnp.int32),
            pltpu.VMEM((SB, CHUNK), jnp.int32),
            pltpu.VMEM((CHUNK, D), jnp.float32),
            pltpu.VMEM((CHUNK, D), jnp.float32),
            pltpu.VMEM_SHARED((NPAD, D), jnp.float32),
            pltpu.SemaphoreType.DMA,
            pltpu.SemaphoreType.DMA,
        ],
    )(x, src, dst, zero)


def _linear_body(h2_ref, w_ref, b_ref, o_ref):
    h = h2_ref[0] + h2_ref[1]
    o_ref[...] = lax.dot_general(
        h, w_ref[...], (((1,), (1,)), ((), ())),
        preferred_element_type=jnp.float32) + b_ref[...]


@jax.jit
def _linear_tc(h2, W, b2):
    blk = 1000
    grid = N_NODES // blk
    return pl.pallas_call(
        _linear_body,
        grid=(grid,),
        in_specs=[
            pl.BlockSpec((NC, blk, D), lambda i: (0, i, 0)),
            pl.BlockSpec((D, D), lambda i: (0, 0)),
            pl.BlockSpec((1, D), lambda i: (0, 0)),
        ],
        out_specs=pl.BlockSpec((blk, D), lambda i: (i, 0)),
        out_shape=jax.ShapeDtypeStruct((N_NODES, D), jnp.float32),
    )(h2, W, b2)


def kernel(inputs, edge_index, W, b):
    n_pad = E_PAD - N_EDGES
    src = jnp.concatenate(
        [edge_index[0], jnp.zeros((n_pad,), jnp.int32)]
    ).reshape(NW, NSB, SB, CHUNK)
    dst = jnp.concatenate(
        [edge_index[1], jnp.full((n_pad,), NPAD - 1, jnp.int32)]
    ).reshape(NW, NSB, SB, CHUNK)
    zero = jnp.zeros((NPAD, D), jnp.float32)
    h2 = _segment_sum_sc(inputs, src, dst, zero)
    return _linear_tc(h2, W, b.reshape(1, D))


# speedup vs baseline: 1.0008x; 1.0008x over previous
import jax
import jax.numpy as jnp
from jax import lax
from jax.experimental import pallas as pl
from jax.experimental.pallas import tpu as pltpu
from jax.experimental.pallas import tpu_sc as plsc

N_NODES = 10000
N_EDGES = 320000
D = 128

NC = 2
NS = 16
NW = NC * NS

CHUNK = 128
SB = 8
NSB = 10
NCHUNK = SB * NSB
E_PAD = NW * NCHUNK * CHUNK
NPAD = 10112
ROWS_PER_TILE = NPAD // NS


def _scatter_gather_kernel(x_hbm, src_hbm, dst_hbm, zero_hbm, h2_hbm,
                           sidx0, sidx1, didx0, didx1, buf0, buf1, acc,
                           gsem0, gsem1):
    c = lax.axis_index("c")
    s = lax.axis_index("s")
    wid = s * NC + c

    sidx = (sidx0, sidx1)
    didx = (didx0, didx1)
    bufs = (buf0, buf1)
    gsem = (gsem0, gsem1)

    def gather(idx_row, b):
        return pltpu.async_copy(x_hbm.at[idx_row], bufs[b], gsem[b])

    def gather_wait(b):
        pltpu.make_async_copy(x_hbm.at[sidx0.at[0]], bufs[b], gsem[b]).wait()

    r0 = s * ROWS_PER_TILE
    pltpu.sync_copy(zero_hbm.at[pl.ds(r0, ROWS_PER_TILE)],
                    acc.at[pl.ds(r0, ROWS_PER_TILE)])

    pltpu.sync_copy(src_hbm.at[wid, 0], sidx0)
    pltpu.sync_copy(dst_hbm.at[wid, 0], didx0)
    plsc.subcore_barrier()
    gather(sidx0.at[0], 0)
    gather(sidx0.at[1], 1)

    def body(i, carry):
        for sbk in range(2):
            sb = 2 * i + sbk
            sbn = lax.rem(sb + 1, NSB)
            pltpu.sync_copy(src_hbm.at[wid, sbn], sidx[1 - sbk])
            pltpu.sync_copy(dst_hbm.at[wid, sbn], didx[1 - sbk])
            for k in range(SB):
                b = k % 2
                gather_wait(b)
                pltpu.sync_copy(bufs[b], acc.at[didx[sbk].at[k]], add=True)
                if k < SB - 2:
                    gather(sidx[sbk].at[k + 2], b)
                else:
                    gather(sidx[1 - sbk].at[k + 2 - SB], b)
        return carry

    lax.fori_loop(0, NSB // 2, body, 0)

    gather_wait(0)
    gather_wait(1)

    plsc.subcore_barrier()
    pltpu.sync_copy(acc.at[pl.ds(r0, ROWS_PER_TILE)],
                    h2_hbm.at[c, pl.ds(r0, ROWS_PER_TILE)])


@jax.jit
def _segment_sum_sc(x, src, dst, zero):
    mesh = plsc.VectorSubcoreMesh(core_axis_name="c", subcore_axis_name="s")
    return pl.kernel(
        _scatter_gather_kernel,
        out_type=jax.ShapeDtypeStruct((NC, NPAD, D), jnp.float32),
        mesh=mesh,
        scratch_types=[
            pltpu.VMEM((SB, CHUNK), jnp.int32),
            pltpu.VMEM((SB, CHUNK), jnp.int32),
            pltpu.VMEM((SB, CHUNK), jnp.int32),
            pltpu.VMEM((SB, CHUNK), jnp.int32),
            pltpu.VMEM((CHUNK, D), jnp.float32),
            pltpu.VMEM((CHUNK, D), jnp.float32),
            pltpu.VMEM_SHARED((NPAD, D), jnp.float32),
            pltpu.SemaphoreType.DMA,
            pltpu.SemaphoreType.DMA,
        ],
    )(x, src, dst, zero)


def _linear_body(h2_ref, w_ref, b_ref, o_ref):
    h = h2_ref[0] + h2_ref[1]
    o_ref[...] = lax.dot_general(
        h, w_ref[...], (((1,), (1,)), ((), ())),
        preferred_element_type=jnp.float32) + b_ref[...]


@jax.jit
def _linear_tc(h2, W, b2):
    blk = 1000
    grid = N_NODES // blk
    return pl.pallas_call(
        _linear_body,
        grid=(grid,),
        in_specs=[
            pl.BlockSpec((NC, blk, D), lambda i: (0, i, 0)),
            pl.BlockSpec((D, D), lambda i: (0, 0)),
            pl.BlockSpec((1, D), lambda i: (0, 0)),
        ],
        out_specs=pl.BlockSpec((blk, D), lambda i: (i, 0)),
        out_shape=jax.ShapeDtypeStruct((N_NODES, D), jnp.float32),
    )(h2, W, b2)


def kernel(inputs, edge_index, W, b):
    n_pad = E_PAD - N_EDGES
    src = jnp.concatenate(
        [edge_index[0], jnp.zeros((n_pad,), jnp.int32)]
    ).reshape(NW, NSB, SB, CHUNK)
    dst = jnp.concatenate(
        [edge_index[1], jnp.full((n_pad,), NPAD - 1, jnp.int32)]
    ).reshape(NW, NSB, SB, CHUNK)
    zero = jnp.zeros((NPAD, D), jnp.float32)
    h2 = _segment_sum_sc(inputs, src, dst, zero)
    return _linear_tc(h2, W, b.reshape(1, D))
